# baseline (speedup 1.0000x reference)
import functools

import jax
import jax.numpy as jnp
from jax import lax
from jax.experimental import pallas as pl
from jax.experimental.pallas import tpu as pltpu
from jax.experimental.pallas import tpu_sc as plsc

D = 16
T_REAL = 200
TPAD = 256
NG = 208
ROW_BLK = 8192
BLKC = 1024
NW = 32


def _prep_body(t_ref, w1b_ref, b1r_ref, w2b_ref, k_ref):
    rows = t_ref[...]
    h = jnp.maximum(
        jnp.dot(rows, w1b_ref[...], preferred_element_type=jnp.float32)
        + b1r_ref[...], 0.0)
    k_ref[...] = jnp.dot(h, w2b_ref[...], preferred_element_type=jnp.float32)


def _prep(iid128, W1b, b1r, W2b):
    nrow = iid128.shape[0]
    blk = ROW_BLK * D // 128
    nblk = nrow // blk
    return pl.pallas_call(
        _prep_body,
        grid=(nblk,),
        in_specs=[
            pl.BlockSpec((blk, 128), lambda i: (i, 0)),
            pl.BlockSpec((128, 128), lambda i: (0, 0)),
            pl.BlockSpec((1, 128), lambda i: (0, 0)),
            pl.BlockSpec((128, 128), lambda i: (0, 0)),
        ],
        out_specs=pl.BlockSpec((blk, 128), lambda i: (i, 0)),
        out_shape=jax.ShapeDtypeStruct((nrow, 128), jnp.float32),
    )(iid128, W1b, b1r, W2b)


def _vbroadcast(vec, idxvec):
    dnums = lax.GatherDimensionNumbers(
        offset_dims=(), collapsed_slice_dims=(0,), start_index_map=(0,))
    return lax.gather(vec, idxvec[:, None], dnums, (1,),
                      mode=lax.GatherScatterMode.PROMISE_IN_BOUNDS)


_SC_PARAMS = dict(
    compiler_params=None,
)


def _sc_main_body(seqp, ktab, siid, his_o,
                  idx_f, kidx_f, rows_v, k_v, his_v,
                  sem_s0, sem_s1, sem_g0, sem_g1, bpw):
    c = lax.axis_index("c")
    s = lax.axis_index("s")
    wid = s * 2 + c
    base = wid * bpw
    sem_s = (sem_s0, sem_s1)
    sem_g = (sem_g0, sem_g1)

    lane = lax.iota(jnp.int32, 16)
    lane_full = [lax.broadcast(jnp.int32(l), (16,)) for l in range(16)]

    def _seq_start(i, p):
        off = pl.multiple_of((base + i) * TPAD, 16)
        pltpu.async_copy(seqp.at[pl.ds(off, NG)], idx_f.at[p], sem_s[p])

    def _seq_wait(p):
        pltpu.make_async_copy(
            seqp.at[pl.ds(0, NG)], idx_f.at[p], sem_s[p]).wait()

    def _gather_start(p):
        for v in range(13):
            kidx_f[p, pl.ds(v * 16, 16)] = (
                idx_f[p, pl.ds(v * 16, 16)] * jnp.int32(16))
        ia = idx_f.at[p, pl.ds(0, 128)]
        ib = idx_f.at[p, pl.ds(128, 80)]
        ka = kidx_f.at[p, pl.ds(0, 128)]
        kb = kidx_f.at[p, pl.ds(128, 80)]
        pltpu.async_copy(siid.at[ia], rows_v.at[p, pl.ds(0, 128)], sem_g[p])
        pltpu.async_copy(siid.at[ib], rows_v.at[p, pl.ds(128, 80)], sem_g[p])
        pltpu.async_copy(ktab.at[ka], k_v.at[p, pl.ds(0, 128)], sem_g[p])
        pltpu.async_copy(ktab.at[kb], k_v.at[p, pl.ds(128, 80)], sem_g[p])

    def _gather_wait(p):
        ia = idx_f.at[p, pl.ds(0, 128)]
        ib = idx_f.at[p, pl.ds(128, 80)]
        ka = kidx_f.at[p, pl.ds(0, 128)]
        kb = kidx_f.at[p, pl.ds(128, 80)]
        pltpu.make_async_copy(
            siid.at[ia], rows_v.at[p, pl.ds(0, 128)], sem_g[p]).wait()
        pltpu.make_async_copy(
            siid.at[ib], rows_v.at[p, pl.ds(128, 80)], sem_g[p]).wait()
        pltpu.make_async_copy(
            ktab.at[ka], k_v.at[p, pl.ds(0, 128)], sem_g[p]).wait()
        pltpu.make_async_copy(
            ktab.at[kb], k_v.at[p, pl.ds(128, 80)], sem_g[p]).wait()

    def _stage(i, cur):
        nxt = 1 - cur
        _gather_wait(cur)

        @pl.when(i + 1 < bpw)
        def _():
            _seq_wait(nxt)
            _gather_start(nxt)

        ivs = [idx_f[cur, pl.ds(v * 16, 16)] for v in range(13)]

        @pl.when(i + 2 < bpw)
        def _():
            _seq_start(i + 2, cur)

        tvs = []
        for v in range(13):
            kv = k_v[cur, pl.ds(v * 16, 16)]
            tv = jnp.where(ivs[v] == 0, kv - 1e8, kv)
            if (v + 1) * 16 > T_REAL:
                tv = jnp.where(lane < T_REAL - v * 16, tv, -1e30)
            tvs.append(tv)
        mv = tvs[0]
        for tv in tvs[1:]:
            mv = jnp.maximum(mv, tv)
        m = jnp.max(mv)
        evs = []
        svec = None
        for v in range(13):
            ev = jnp.exp(tvs[v] - m)
            evs.append(ev)
            svec = ev if svec is None else svec + ev
        rvec = 1.0 / lax.broadcast(jnp.sum(svec), (16,))

        his = jnp.zeros((16,), jnp.float32)
        for v in range(13):
            ev = evs[v]
            for l in range(16):
                w = _vbroadcast(ev, lane_full[l])
                his = his + rows_v[cur, v * 16 + l] * w
        his_v[i] = his * rvec

    _seq_start(0, 0)
    _seq_wait(0)
    _gather_start(0)
    _seq_start(1, 1)

    def pair_body(j, carry):
        _stage(2 * j, 0)
        _stage(2 * j + 1, 1)
        return carry

    lax.fori_loop(0, bpw // 2, pair_body, 0)
    pltpu.sync_copy(his_v, his_o.at[pl.ds(base, bpw)])


def _sc_main(seqp, ktab, siid, B):
    bpw = B // NW
    mesh = plsc.VectorSubcoreMesh(core_axis_name="c", subcore_axis_name="s")
    f32, i32 = jnp.float32, jnp.int32
    scratch = [
        pltpu.VMEM((2, NG), i32),
        pltpu.VMEM((2, NG), i32),
        pltpu.VMEM((2, NG, D), f32),
        pltpu.VMEM((2, NG), f32),
        pltpu.VMEM((bpw, D), f32),
        pltpu.SemaphoreType.DMA,
        pltpu.SemaphoreType.DMA,
        pltpu.SemaphoreType.DMA,
        pltpu.SemaphoreType.DMA,
    ]
    params = pltpu.CompilerParams(use_tc_tiling_on_sc=False,
                                  needs_layout_passes=False)
    return pl.kernel(functools.partial(_sc_main_body, bpw=bpw),
                     out_type=jax.ShapeDtypeStruct((B, D), f32),
                     mesh=mesh, compiler_params=params,
                     scratch_types=scratch)(seqp, ktab, siid)


def _sc_pair_body(uidx, iidx, suid, tiid, uro, iro,
                  uidx_v, iidx_v, urows_v, irows_v, sem_u, bpw):
    c = lax.axis_index("c")
    s = lax.axis_index("s")
    wid = s * 2 + c
    base = wid * bpw
    nch = bpw // 128

    pltpu.sync_copy(uidx.at[pl.ds(base, bpw)], uidx_v)
    pltpu.sync_copy(iidx.at[pl.ds(base, bpw)], iidx_v)
    descs = []
    for j in range(nch):
        descs.append(pltpu.async_copy(
            suid.at[uidx_v.at[pl.ds(j * 128, 128)]],
            urows_v.at[pl.ds(j * 128, 128)], sem_u))
        descs.append(pltpu.async_copy(
            tiid.at[iidx_v.at[pl.ds(j * 128, 128)]],
            irows_v.at[pl.ds(j * 128, 128)], sem_u))
    for d in descs:
        d.wait()
    pltpu.sync_copy(urows_v, uro.at[pl.ds(base, bpw)])
    pltpu.sync_copy(irows_v, iro.at[pl.ds(base, bpw)])


def _sc_pair(uidx, iidx, suid, tiid, B):
    bpw = B // NW
    mesh = plsc.VectorSubcoreMesh(core_axis_name="c", subcore_axis_name="s")
    f32, i32 = jnp.float32, jnp.int32
    out_type = (
        jax.ShapeDtypeStruct((B, D), f32),
        jax.ShapeDtypeStruct((B, D), f32),
    )
    scratch = [
        pltpu.VMEM((bpw,), i32),
        pltpu.VMEM((bpw,), i32),
        pltpu.VMEM((bpw, D), f32),
        pltpu.VMEM((bpw, D), f32),
        pltpu.SemaphoreType.DMA,
    ]
    params = pltpu.CompilerParams(use_tc_tiling_on_sc=False,
                                  needs_layout_passes=False)
    return pl.kernel(functools.partial(_sc_pair_body, bpw=bpw),
                     out_type=out_type, mesh=mesh, compiler_params=params,
                     scratch_types=scratch)(uidx, iidx, suid, tiid)


def _final_body(his_ref, ur_ref, ir_ref, w3_ref, b3_ref, w4_ref, b4_ref,
                out_ref, emb_ref, ls_ref):
    his = his_ref[...]
    a = jnp.maximum(
        jnp.dot(his, w3_ref[...], preferred_element_type=jnp.float32)
        + b3_ref[...], 0.0)
    dec = (jnp.dot(a, w4_ref[...], preferred_element_type=jnp.float32)
           + b4_ref[...])
    ur = ur_ref[...]
    ue = ur[:, 0:1] * dec[:, 0:D]
    for k in range(1, D):
        ue = ue + ur[:, k:k + 1] * dec[:, k * D:(k + 1) * D]
    ir = ir_ref[...]
    out_ref[...] = jnp.sum(ue * ir, axis=1)
    emb_ref[...] = jnp.stack([ue, ir], axis=1)
    part = (jnp.sum(ue * ue) + jnp.sum(ir * ir)).reshape(1, 1)

    @pl.when(pl.program_id(0) == 0)
    def _init():
        ls_ref[...] = jnp.zeros_like(ls_ref)

    ls_ref[...] += part


def _final(his, urows, irows, W3, b3, W4, b4, B):
    grid = (B // BLKC,)
    M = W3.shape[1]
    return pl.pallas_call(
        _final_body,
        grid=grid,
        in_specs=[
            pl.BlockSpec((BLKC, D), lambda i: (i, 0)),
            pl.BlockSpec((BLKC, D), lambda i: (i, 0)),
            pl.BlockSpec((BLKC, D), lambda i: (i, 0)),
            pl.BlockSpec((D, M), lambda i: (0, 0)),
            pl.BlockSpec((1, M), lambda i: (0, 0)),
            pl.BlockSpec((M, D * D), lambda i: (0, 0)),
            pl.BlockSpec((1, D * D), lambda i: (0, 0)),
        ],
        out_specs=[
            pl.BlockSpec((BLKC,), lambda i: (i,)),
            pl.BlockSpec((BLKC, 2, D), lambda i: (i, 0, 0)),
            pl.BlockSpec((1, 1), lambda i: (0, 0)),
        ],
        out_shape=[
            jax.ShapeDtypeStruct((B,), jnp.float32),
            jax.ShapeDtypeStruct((B, 2, D), jnp.float32),
            jax.ShapeDtypeStruct((1, 1), jnp.float32),
        ],
    )(his, urows, irows, W3, b3.reshape(1, M), W4, b4.reshape(1, D * D))


def _linear128(table, npad):
    n = table.shape[0]
    return jnp.pad(table, ((0, npad - n), (0, 0))).reshape(npad * D // 128,
                                                           128)


def kernel(x, src_uid, src_iid, tgt_iid, W1, b1, W2, W3, b3, W4, b4):
    B = x.shape[0]
    n = src_iid.shape[0]
    nblk = (n + ROW_BLK - 1) // ROW_BLK
    npad = nblk * ROW_BLK

    seq = x[:, 2:]
    padv = ((jnp.arange(B, dtype=jnp.int32)[:, None] * (TPAD - T_REAL)
             + jnp.arange(TPAD - T_REAL, dtype=jnp.int32)[None, :] + 1)
            % jnp.int32(1000000))
    seqp = jnp.concatenate([seq, padv], axis=1)
    uidx = x[:, 0]
    iidx = x[:, 1]

    iid128 = _linear128(src_iid, npad)
    uid128 = _linear128(src_uid, npad)
    tid128 = _linear128(tgt_iid, npad)

    eye8 = jnp.eye(8, dtype=jnp.float32)
    W1b = jnp.kron(eye8, W1)
    W2b = jnp.kron(eye8, jnp.tile(W2, (1, D)))
    b1r = jnp.tile(b1, 8).reshape(1, 128)

    ktabw = _prep(iid128, W1b, b1r, W2b)

    his = _sc_main(seqp.reshape(B * TPAD),
                   ktabw.reshape(npad * D),
                   iid128.reshape(npad, D), B)
    urows, irows = _sc_pair(uidx, iidx,
                            uid128.reshape(npad, D),
                            tid128.reshape(npad, D), B)

    output, emb, ls = _final(his, urows, irows, W3, b3, W4, b4, B)

    emb_loss = jnp.sqrt(ls[0, 0]) / B
    return (output, emb_loss, emb)


# speedup vs baseline: 1.4012x; 1.4012x over previous
import functools

import jax
import jax.numpy as jnp
from jax import lax
from jax.experimental import pallas as pl
from jax.experimental.pallas import tpu as pltpu
from jax.experimental.pallas import tpu_sc as plsc

D = 16
T_REAL = 200
TPAD = 256
NG = 208
ROW_BLK = 8192
BLKC = 1024
NW = 32


def _ktable_body(tab_ref, w1_ref, b1_ref, w2t_ref, o_ref):
    rows = tab_ref[...]
    h = jnp.maximum(
        jnp.dot(rows, w1_ref[...], preferred_element_type=jnp.float32)
        + b1_ref[...], 0.0)
    o_ref[...] = jnp.sum(h * w2t_ref[...], axis=1)


def _ktable(src_iid, W1, b1, W2):
    n = src_iid.shape[0]
    nblk = (n + ROW_BLK - 1) // ROW_BLK
    npad = nblk * ROW_BLK
    return pl.pallas_call(
        _ktable_body,
        grid=(nblk,),
        in_specs=[
            pl.BlockSpec((ROW_BLK, D), lambda i: (i, 0)),
            pl.BlockSpec((D, D), lambda i: (0, 0)),
            pl.BlockSpec((1, D), lambda i: (0, 0)),
            pl.BlockSpec((1, D), lambda i: (0, 0)),
        ],
        out_specs=pl.BlockSpec((ROW_BLK,), lambda i: (i,)),
        out_shape=jax.ShapeDtypeStruct((npad,), jnp.float32),
    )(src_iid, W1, b1.reshape(1, D), W2.reshape(1, D))


def _vbroadcast(vec, idxvec):
    dnums = lax.GatherDimensionNumbers(
        offset_dims=(), collapsed_slice_dims=(0,), start_index_map=(0,))
    return lax.gather(vec, idxvec[:, None], dnums, (1,),
                      mode=lax.GatherScatterMode.PROMISE_IN_BOUNDS)


def _sc_main_body(seqp, ktab, siid, his_o,
                  idx_f, rows_v, k_v, his_v,
                  sem_s0, sem_s1, sem_g0, sem_g1, bpw):
    c = lax.axis_index("c")
    s = lax.axis_index("s")
    wid = s * 2 + c
    base = wid * bpw
    sem_s = (sem_s0, sem_s1)
    sem_g = (sem_g0, sem_g1)

    lane = lax.iota(jnp.int32, 16)
    lane_full = [lax.broadcast(jnp.int32(l), (16,)) for l in range(16)]

    def _seq_start(i, p):
        off = pl.multiple_of((base + i) * TPAD, 16)
        pltpu.async_copy(seqp.at[pl.ds(off, NG)], idx_f.at[p], sem_s[p])

    def _seq_wait(p):
        pltpu.make_async_copy(
            seqp.at[pl.ds(0, NG)], idx_f.at[p], sem_s[p]).wait()

    def _gather_start(p):
        ia = idx_f.at[p, pl.ds(0, 128)]
        ib = idx_f.at[p, pl.ds(128, 80)]
        pltpu.async_copy(siid.at[ia], rows_v.at[p, pl.ds(0, 128)], sem_g[p])
        pltpu.async_copy(siid.at[ib], rows_v.at[p, pl.ds(128, 80)], sem_g[p])
        pltpu.async_copy(ktab.at[ia], k_v.at[p, pl.ds(0, 128)], sem_g[p])
        pltpu.async_copy(ktab.at[ib], k_v.at[p, pl.ds(128, 80)], sem_g[p])

    def _gather_wait(p):
        ia = idx_f.at[p, pl.ds(0, 128)]
        ib = idx_f.at[p, pl.ds(128, 80)]
        pltpu.make_async_copy(
            siid.at[ia], rows_v.at[p, pl.ds(0, 128)], sem_g[p]).wait()
        pltpu.make_async_copy(
            siid.at[ib], rows_v.at[p, pl.ds(128, 80)], sem_g[p]).wait()
        pltpu.make_async_copy(
            ktab.at[ia], k_v.at[p, pl.ds(0, 128)], sem_g[p]).wait()
        pltpu.make_async_copy(
            ktab.at[ib], k_v.at[p, pl.ds(128, 80)], sem_g[p]).wait()

    def _stage(i, cur):
        nxt = 1 - cur
        _gather_wait(cur)

        @pl.when(i + 1 < bpw)
        def _():
            _seq_wait(nxt)
            _gather_start(nxt)

        ivs = [idx_f[cur, pl.ds(v * 16, 16)] for v in range(13)]

        @pl.when(i + 2 < bpw)
        def _():
            _seq_start(i + 2, cur)

        tvs = []
        for v in range(13):
            kv = k_v[cur, pl.ds(v * 16, 16)]
            tv = jnp.where(ivs[v] == 0, kv - 1e8, kv)
            if (v + 1) * 16 > T_REAL:
                tv = jnp.where(lane < T_REAL - v * 16, tv, -1e30)
            tvs.append(tv)
        mv = tvs[0]
        for tv in tvs[1:]:
            mv = jnp.maximum(mv, tv)
        m = jnp.max(mv)
        evs = []
        svec = None
        for v in range(13):
            ev = jnp.exp(tvs[v] - m)
            evs.append(ev)
            svec = ev if svec is None else svec + ev
        rvec = 1.0 / lax.broadcast(jnp.sum(svec), (16,))

        his = jnp.zeros((16,), jnp.float32)
        for v in range(13):
            ev = evs[v]
            for l in range(16):
                w = _vbroadcast(ev, lane_full[l])
                his = his + rows_v[cur, v * 16 + l] * w
        his_v[i] = his * rvec

    _seq_start(0, 0)
    _seq_wait(0)
    _gather_start(0)
    _seq_start(1, 1)

    def pair_body(j, carry):
        _stage(2 * j, 0)
        _stage(2 * j + 1, 1)
        return carry

    lax.fori_loop(0, bpw // 2, pair_body, 0)
    pltpu.sync_copy(his_v, his_o.at[pl.ds(base, bpw)])


def _sc_main(seqp, ktab, siid, B):
    bpw = B // NW
    mesh = plsc.VectorSubcoreMesh(core_axis_name="c", subcore_axis_name="s")
    f32, i32 = jnp.float32, jnp.int32
    scratch = [
        pltpu.VMEM((2, NG), i32),
        pltpu.VMEM((2, NG, D), f32),
        pltpu.VMEM((2, NG), f32),
        pltpu.VMEM((bpw, D), f32),
        pltpu.SemaphoreType.DMA,
        pltpu.SemaphoreType.DMA,
        pltpu.SemaphoreType.DMA,
        pltpu.SemaphoreType.DMA,
    ]
    params = pltpu.CompilerParams(use_tc_tiling_on_sc=False,
                                  needs_layout_passes=False)
    return pl.kernel(functools.partial(_sc_main_body, bpw=bpw),
                     out_type=jax.ShapeDtypeStruct((B, D), f32),
                     mesh=mesh, compiler_params=params,
                     scratch_types=scratch)(seqp, ktab, siid)


def _sc_pair_body(uidx, iidx, suid, tiid, uro, iro,
                  uidx_v, iidx_v, urows_v, irows_v, sem_u, bpw):
    c = lax.axis_index("c")
    s = lax.axis_index("s")
    wid = s * 2 + c
    base = wid * bpw
    nch = bpw // 128

    pltpu.sync_copy(uidx.at[pl.ds(base, bpw)], uidx_v)
    pltpu.sync_copy(iidx.at[pl.ds(base, bpw)], iidx_v)
    descs = []
    for j in range(nch):
        descs.append(pltpu.async_copy(
            suid.at[uidx_v.at[pl.ds(j * 128, 128)]],
            urows_v.at[pl.ds(j * 128, 128)], sem_u))
        descs.append(pltpu.async_copy(
            tiid.at[iidx_v.at[pl.ds(j * 128, 128)]],
            irows_v.at[pl.ds(j * 128, 128)], sem_u))
    for d in descs:
        d.wait()
    pltpu.sync_copy(urows_v, uro.at[pl.ds(base, bpw)])
    pltpu.sync_copy(irows_v, iro.at[pl.ds(base, bpw)])


def _sc_pair(uidx, iidx, suid, tiid, B):
    bpw = B // NW
    mesh = plsc.VectorSubcoreMesh(core_axis_name="c", subcore_axis_name="s")
    f32, i32 = jnp.float32, jnp.int32
    out_type = (
        jax.ShapeDtypeStruct((B, D), f32),
        jax.ShapeDtypeStruct((B, D), f32),
    )
    scratch = [
        pltpu.VMEM((bpw,), i32),
        pltpu.VMEM((bpw,), i32),
        pltpu.VMEM((bpw, D), f32),
        pltpu.VMEM((bpw, D), f32),
        pltpu.SemaphoreType.DMA,
    ]
    params = pltpu.CompilerParams(use_tc_tiling_on_sc=False,
                                  needs_layout_passes=False)
    return pl.kernel(functools.partial(_sc_pair_body, bpw=bpw),
                     out_type=out_type, mesh=mesh, compiler_params=params,
                     scratch_types=scratch)(uidx, iidx, suid, tiid)


def _final_body(his_ref, ur_ref, ir_ref, w3_ref, b3_ref, w4_ref, b4_ref,
                out_ref, emb_ref, ls_ref):
    his = his_ref[...]
    a = jnp.maximum(
        jnp.dot(his, w3_ref[...], preferred_element_type=jnp.float32)
        + b3_ref[...], 0.0)
    dec = (jnp.dot(a, w4_ref[...], preferred_element_type=jnp.float32)
           + b4_ref[...])
    ur = ur_ref[...]
    ue = ur[:, 0:1] * dec[:, 0:D]
    for k in range(1, D):
        ue = ue + ur[:, k:k + 1] * dec[:, k * D:(k + 1) * D]
    ir = ir_ref[...]
    out_ref[...] = jnp.sum(ue * ir, axis=1)
    emb_ref[...] = jnp.stack([ue, ir], axis=1)
    part = (jnp.sum(ue * ue) + jnp.sum(ir * ir)).reshape(1, 1)

    @pl.when(pl.program_id(0) == 0)
    def _init():
        ls_ref[...] = jnp.zeros_like(ls_ref)

    ls_ref[...] += part


def _final(his, urows, irows, W3, b3, W4, b4, B):
    grid = (B // BLKC,)
    M = W3.shape[1]
    return pl.pallas_call(
        _final_body,
        grid=grid,
        in_specs=[
            pl.BlockSpec((BLKC, D), lambda i: (i, 0)),
            pl.BlockSpec((BLKC, D), lambda i: (i, 0)),
            pl.BlockSpec((BLKC, D), lambda i: (i, 0)),
            pl.BlockSpec((D, M), lambda i: (0, 0)),
            pl.BlockSpec((1, M), lambda i: (0, 0)),
            pl.BlockSpec((M, D * D), lambda i: (0, 0)),
            pl.BlockSpec((1, D * D), lambda i: (0, 0)),
        ],
        out_specs=[
            pl.BlockSpec((BLKC,), lambda i: (i,)),
            pl.BlockSpec((BLKC, 2, D), lambda i: (i, 0, 0)),
            pl.BlockSpec((1, 1), lambda i: (0, 0)),
        ],
        out_shape=[
            jax.ShapeDtypeStruct((B,), jnp.float32),
            jax.ShapeDtypeStruct((B, 2, D), jnp.float32),
            jax.ShapeDtypeStruct((1, 1), jnp.float32),
        ],
    )(his, urows, irows, W3, b3.reshape(1, M), W4, b4.reshape(1, D * D))


def kernel(x, src_uid, src_iid, tgt_iid, W1, b1, W2, W3, b3, W4, b4):
    B = x.shape[0]

    seq = x[:, 2:]
    padv = ((jnp.arange(B, dtype=jnp.int32)[:, None] * (TPAD - T_REAL)
             + jnp.arange(TPAD - T_REAL, dtype=jnp.int32)[None, :] + 1)
            % jnp.int32(1000000))
    seqp = jnp.concatenate([seq, padv], axis=1)
    uidx = x[:, 0]
    iidx = x[:, 1]

    ktab = _ktable(src_iid, W1, b1, W2)
    his = _sc_main(seqp.reshape(B * TPAD), ktab, src_iid, B)
    urows, irows = _sc_pair(uidx, iidx, src_uid, tgt_iid, B)
    output, emb, ls = _final(his, urows, irows, W3, b3, W4, b4, B)

    emb_loss = jnp.sqrt(ls[0, 0]) / B
    return (output, emb_loss, emb)
